# Initial kernel scaffold; baseline (speedup 1.0000x reference)
import jax
import jax.numpy as jnp
from jax.experimental import pallas as pl


def kernel(local_memory_embeddings, local_memory_index, batch_embeddings, batch_indexes):
    raise NotImplementedError("write your pallas kernel here")


# speedup vs baseline: 8.7912x; 8.7912x over previous
import functools

import jax
import jax.numpy as jnp
from jax import lax
from jax.experimental import pallas as pl
from jax.experimental.pallas import tpu as pltpu
from jax.experimental.pallas import tpu_sc as plsc

N = 250000
D = 128
B = 16384
L = 16
NC = 2
NS = 16
NW = NC * NS

R = 7824
WSZ = R + L
CH = 256
NCHMAX = (R + CH - 1) // CH
FLAT = NCHMAX * CH + CH

_mesh = plsc.VectorSubcoreMesh(
    core_axis_name="c", subcore_axis_name="s", num_cores=NC, num_subcores=NS
)


@functools.partial(
    pl.kernel,
    out_type=(),
    mesh=_mesh,
    compiler_params=pltpu.CompilerParams(needs_layout_passes=False),
    scratch_types=[
        pltpu.VMEM((B,), jnp.int32),
        pltpu.VMEM((WSZ,), jnp.int32),
        pltpu.VMEM((FLAT,), jnp.int32),
        pltpu.VMEM((FLAT,), jnp.int32),
        pltpu.VMEM((CH, D), jnp.float32),
        pltpu.VMEM((2 * L,), jnp.int32),
        pltpu.SemaphoreType.DMA,
    ],
)
def _sc_update(bemb, bidxh, memh, idxh,
               bidx_v, winner_v, jflat_v, dflat_v, rowbuf_v, tmp_v, sem):
    wid = lax.axis_index("s") * NC + lax.axis_index("c")
    lo = wid * R
    hi = jnp.minimum(lo + R, N)
    iota = lax.broadcasted_iota(jnp.int32, (L,), 0)

    pltpu.sync_copy(bidxh, bidx_v)

    neg1 = jnp.full((L,), -1, jnp.int32)
    def _init(i, _):
        winner_v[pl.ds(i * L, L)] = neg1
        return 0
    lax.fori_loop(0, WSZ // L, _init, 0, unroll=4)
    tmp_v[pl.ds(L, L)] = neg1

    def _scan(g, _):
        d = bidx_v[pl.ds(g * L, L)]
        j = g * L + iota
        inr = (d >= lo) & (d < hi)
        ddl = jnp.where(inr, d - lo, R)
        key = ddl * B + j
        skey = plsc.sort_key_val(key, key)[0]
        sd = lax.shift_right_logical(skey, 14)
        sj = skey & (B - 1)
        tmp_v[pl.ds(0, L)] = sd
        nxt = tmp_v[pl.ds(1, L)]
        keep = sd != nxt
        plsc.store_scatter(winner_v, [sd], sj, mask=keep)
        return 0
    lax.fori_loop(0, B // L, _scan, 0)

    def _compact(g, carry):
        cnt, best = carry
        w = winner_v[pl.ds(g * L, L)]
        m = w >= 0
        dst = lo + g * L + iota
        plsc.store_compressed(jflat_v.at[pl.ds(cnt, L)], w, mask=m)
        plsc.store_compressed(dflat_v.at[pl.ds(cnt, L)], dst, mask=m)
        popc = plsc.all_reduce_population_count(m)
        npop = popc if popc.ndim == 0 else jnp.max(popc)
        enc = jnp.where(m, (g * L + iota) * B + w, -1)
        best = jnp.maximum(best, jnp.max(enc))
        return cnt + npop, best
    cnt, best = lax.fori_loop(0, R // L, _compact, (0, -1))

    @pl.when(cnt > 0)
    def _move():
        pad_j = jnp.full((L,), best & (B - 1), jnp.int32)
        pad_d = jnp.full((L,), lo + lax.shift_right_logical(best, 14), jnp.int32)
        def _pad(t, _):
            jflat_v[pl.ds(cnt + t * L, L)] = pad_j
            dflat_v[pl.ds(cnt + t * L, L)] = pad_d
            return 0
        lax.fori_loop(0, CH // L, _pad, 0, unroll=4)

        nch = (cnt + CH - 1) // CH

        def _chunk(ci, _):
            dlist = dflat_v.at[pl.ds(ci * CH, CH)]
            pltpu.async_copy(
                bemb.at[jflat_v.at[pl.ds(ci * CH, CH)]], rowbuf_v, sem
            ).wait()
            pltpu.async_copy(rowbuf_v, memh.at[dlist], sem).wait()
            pltpu.async_copy(dlist, idxh.at[dlist], sem).wait()
            return 0
        lax.fori_loop(0, nch, _chunk, 0)


def kernel(local_memory_embeddings, local_memory_index, batch_embeddings, batch_indexes):
    mem_ref = jax.new_ref(local_memory_embeddings)
    idx_ref = jax.new_ref(local_memory_index)
    _sc_update(batch_embeddings, batch_indexes, mem_ref, idx_ref)
    return mem_ref[...], idx_ref[...]


# speedup vs baseline: 9.1422x; 1.0399x over previous
import functools

import jax
import jax.numpy as jnp
from jax import lax
from jax.experimental import pallas as pl
from jax.experimental.pallas import tpu as pltpu
from jax.experimental.pallas import tpu_sc as plsc

N = 250000
D = 128
B = 16384
L = 16
NC = 2
NS = 16
NW = NC * NS

R = 7824
WSZ = R + L
CH = 256
NCHMAX = (R + CH - 1) // CH
FLAT = NCHMAX * CH + CH

_mesh = plsc.VectorSubcoreMesh(
    core_axis_name="c", subcore_axis_name="s", num_cores=NC, num_subcores=NS
)


@functools.partial(
    pl.kernel,
    out_type=(),
    mesh=_mesh,
    compiler_params=pltpu.CompilerParams(needs_layout_passes=False),
    scratch_types=[
        pltpu.VMEM((B,), jnp.int32),
        pltpu.VMEM((WSZ,), jnp.int32),
        pltpu.VMEM((FLAT,), jnp.int32),
        pltpu.VMEM((FLAT,), jnp.int32),
        pltpu.VMEM((CH, D), jnp.float32),
        pltpu.SemaphoreType.DMA,
    ],
)
def _sc_update(bemb, bidxh, memh, idxh,
               bidx_v, winner_v, jflat_v, dflat_v, rowbuf_v, sem):
    wid = lax.axis_index("s") * NC + lax.axis_index("c")
    lo = wid * R
    hi = jnp.minimum(lo + R, N)
    iota = lax.broadcasted_iota(jnp.int32, (L,), 0)

    pltpu.sync_copy(bidxh, bidx_v)

    neg1 = jnp.full((L,), -1, jnp.int32)
    def _init(i, _):
        winner_v[pl.ds(i * L, L)] = neg1
        return 0
    lax.fori_loop(0, WSZ // L, _init, 0, unroll=8)

    def _scan(g, _):
        d = bidx_v[pl.ds(g * L, L)]
        j = g * L + iota
        inr = (d >= lo) & (d < hi)
        last = plsc.scan_count(d, mask=inr)[1]
        plsc.store_scatter(
            winner_v, [jnp.where(inr, d - lo, R)], j, mask=last
        )
        return 0
    lax.fori_loop(0, B // L, _scan, 0, unroll=8)

    def _compact(g, carry):
        cnt, bestv = carry
        w = winner_v[pl.ds(g * L, L)]
        m = w >= 0
        dst = lo + g * L + iota
        plsc.store_compressed(jflat_v.at[pl.ds(cnt, L)], w, mask=m)
        plsc.store_compressed(dflat_v.at[pl.ds(cnt, L)], dst, mask=m)
        popc = plsc.all_reduce_population_count(m)
        npop = popc if popc.ndim == 0 else jnp.max(popc)
        enc = jnp.where(m, (g * L + iota) * B + w, -1)
        return cnt + npop, jnp.maximum(bestv, enc)
    cnt, bestv = lax.fori_loop(
        0, R // L, _compact, (0, jnp.full((L,), -1, jnp.int32))
    )
    best = jnp.max(bestv)

    @pl.when(cnt > 0)
    def _move():
        pad_j = jnp.full((L,), best & (B - 1), jnp.int32)
        pad_d = jnp.full((L,), lo + lax.shift_right_logical(best, 14), jnp.int32)
        def _pad(t, _):
            jflat_v[pl.ds(cnt + t * L, L)] = pad_j
            dflat_v[pl.ds(cnt + t * L, L)] = pad_d
            return 0
        lax.fori_loop(0, CH // L, _pad, 0, unroll=4)

        nch = (cnt + CH - 1) // CH

        def _chunk(ci, _):
            dlist = dflat_v.at[pl.ds(ci * CH, CH)]
            pltpu.async_copy(
                bemb.at[jflat_v.at[pl.ds(ci * CH, CH)]], rowbuf_v, sem
            ).wait()
            pltpu.async_copy(rowbuf_v, memh.at[dlist], sem).wait()
            pltpu.async_copy(dlist, idxh.at[dlist], sem).wait()
            return 0
        lax.fori_loop(0, nch, _chunk, 0)


def kernel(local_memory_embeddings, local_memory_index, batch_embeddings, batch_indexes):
    mem_ref = jax.new_ref(local_memory_embeddings)
    idx_ref = jax.new_ref(local_memory_index)
    _sc_update(batch_embeddings, batch_indexes, mem_ref, idx_ref)
    return mem_ref[...], idx_ref[...]
